# baseline (speedup 1.0000x reference)
import functools

import jax
import jax.numpy as jnp
from jax import lax
from jax.experimental import pallas as pl
from jax.experimental.pallas import tpu as pltpu
from jax.experimental.pallas import tpu_sc as plsc

_NC = 2
_NS = 16
_NW = _NC * _NS
_L = 16


def _build_mask_sc(num_rows: int, num_idx: int):
    slab = ((num_rows + _NW - 1) // _NW + _L - 1) // _L * _L
    m_pad = slab * _NW
    mesh = plsc.VectorSubcoreMesh(core_axis_name="c", subcore_axis_name="s")

    @functools.partial(
        pl.kernel,
        out_type=jax.ShapeDtypeStruct((m_pad,), jnp.float32),
        mesh=mesh,
        scratch_types=[
            pltpu.VMEM((num_idx,), jnp.int32),
            pltpu.VMEM((slab,), jnp.float32),
        ],
        compiler_params=pltpu.CompilerParams(needs_layout_passes=False),
    )
    def mask_kernel(idx_hbm, mask_hbm, idx_v, slab_v):
        wid = lax.axis_index("s") * _NC + lax.axis_index("c")
        lo = wid * slab
        pltpu.sync_copy(idx_hbm, idx_v)

        zeros = jnp.zeros((_L,), jnp.float32)

        def zero_body(i, carry):
            slab_v[pl.ds(i * _L, _L)] = zeros
            return carry

        lax.fori_loop(0, slab // _L, zero_body, 0)

        ones = jnp.ones((_L,), jnp.float32)

        def scan_body(i, carry):
            v = idx_v[pl.ds(i * _L, _L)]
            m = (v >= lo) & (v < lo + slab)
            rel = jnp.where(m, v - lo, 0)
            plsc.store_scatter(slab_v, (rel,), ones, mask=m)
            return carry

        lax.fori_loop(0, num_idx // _L, scan_body, 0)

        pltpu.sync_copy(slab_v, mask_hbm.at[pl.ds(lo, slab)])

    return mask_kernel, m_pad


def _select_body(x_ref, m_ref, v_ref, o_ref):
    o_ref[...] = jnp.where(m_ref[...] != 0.0, v_ref[0, 0], x_ref[...])


def _select_tc(x, mask2d, value_f32, blk_rows: int):
    num_rows, d = x.shape
    grid = (pl.cdiv(num_rows, blk_rows),)
    return pl.pallas_call(
        _select_body,
        grid=grid,
        in_specs=[
            pl.BlockSpec((blk_rows, d), lambda i: (i, 0)),
            pl.BlockSpec((blk_rows, 1), lambda i: (i, 0)),
            pl.BlockSpec(memory_space=pltpu.SMEM),
        ],
        out_specs=pl.BlockSpec((blk_rows, d), lambda i: (i, 0)),
        out_shape=jax.ShapeDtypeStruct((num_rows, d), x.dtype),
    )(x, mask2d, value_f32)


def kernel(x, dim, index, value):
    num_rows, _ = x.shape
    num_idx = index.shape[0]
    idx32 = index.astype(jnp.int32)

    mask_fn, m_pad = _build_mask_sc(num_rows, num_idx)
    mask = mask_fn(idx32)
    mask2d = mask.reshape(m_pad, 1)

    value_f32 = jnp.full((1, 1), value, dtype=jnp.float32)
    return _select_tc(x, mask2d, value_f32, blk_rows=1000)


# speedup vs baseline: 1.2210x; 1.2210x over previous
import functools

import jax
import jax.numpy as jnp
from jax import lax
from jax.experimental import pallas as pl
from jax.experimental.pallas import tpu as pltpu
from jax.experimental.pallas import tpu_sc as plsc

_NC = 2
_NS = 16
_NW = _NC * _NS
_L = 16


def _build_mask_sc(num_rows: int, num_idx: int):
    slab = ((num_rows + _NW - 1) // _NW + _L - 1) // _L * _L
    m_pad = slab * _NW
    mesh = plsc.VectorSubcoreMesh(core_axis_name="c", subcore_axis_name="s")

    @functools.partial(
        pl.kernel,
        out_type=jax.ShapeDtypeStruct((m_pad,), jnp.float32),
        mesh=mesh,
        scratch_types=[
            pltpu.VMEM((num_idx,), jnp.int32),
            pltpu.VMEM((slab,), jnp.float32),
        ],
        compiler_params=pltpu.CompilerParams(needs_layout_passes=False),
    )
    def mask_kernel(idx_hbm, mask_hbm, idx_v, slab_v):
        wid = lax.axis_index("s") * _NC + lax.axis_index("c")
        lo = wid * slab
        pltpu.sync_copy(idx_hbm, idx_v)

        zeros = jnp.zeros((_L,), jnp.float32)

        def zero_body(i, carry):
            slab_v[pl.ds(i * _L, _L)] = zeros
            return carry

        lax.fori_loop(0, slab // _L, zero_body, 0)

        ones = jnp.ones((_L,), jnp.float32)

        def scan_body(i, carry):
            v = idx_v[pl.ds(i * _L, _L)]
            m = (v >= lo) & (v < lo + slab)
            rel = jnp.where(m, v - lo, 0)
            plsc.store_scatter(slab_v, (rel,), ones, mask=m)
            return carry

        lax.fori_loop(0, num_idx // _L, scan_body, 0)

        pltpu.sync_copy(slab_v, mask_hbm.at[pl.ds(lo, slab)])

    return mask_kernel, m_pad


def _select_body(x_ref, m_ref, v_ref, o_ref):
    o_ref[...] = jnp.where(m_ref[...] != 0.0, v_ref[0, 0], x_ref[...])


def _select_tc(x, mask2d, value_f32, blk_rows: int):
    num_rows, d = x.shape
    grid = (pl.cdiv(num_rows, blk_rows),)
    return pl.pallas_call(
        _select_body,
        grid=grid,
        in_specs=[
            pl.BlockSpec((blk_rows, d), lambda i: (i, 0)),
            pl.BlockSpec((blk_rows, 1), lambda i: (i, 0)),
            pl.BlockSpec(memory_space=pltpu.SMEM),
        ],
        out_specs=pl.BlockSpec((blk_rows, d), lambda i: (i, 0)),
        out_shape=jax.ShapeDtypeStruct((num_rows, d), x.dtype),
    )(x, mask2d, value_f32)


def kernel(x, dim, index, value):
    num_rows, _ = x.shape
    num_idx = index.shape[0]
    idx32 = index.astype(jnp.int32)

    mask_fn, m_pad = _build_mask_sc(num_rows, num_idx)
    mask = mask_fn(idx32)
    mask2d = mask.reshape(m_pad, 1)

    value_f32 = jnp.full((1, 1), value, dtype=jnp.float32)
    return _select_tc(x, mask2d, value_f32, blk_rows=2000)


# speedup vs baseline: 1.3668x; 1.1194x over previous
import functools

import jax
import jax.numpy as jnp
from jax import lax
from jax.experimental import pallas as pl
from jax.experimental.pallas import tpu as pltpu
from jax.experimental.pallas import tpu_sc as plsc

_NC = 2
_NS = 16
_NW = _NC * _NS
_L = 16


def _build_mask_sc(num_rows: int, num_idx: int):
    slab = ((num_rows + _NW - 1) // _NW + _L - 1) // _L * _L
    m_pad = slab * _NW
    mesh = plsc.VectorSubcoreMesh(core_axis_name="c", subcore_axis_name="s")

    @functools.partial(
        pl.kernel,
        out_type=jax.ShapeDtypeStruct((m_pad,), jnp.float32),
        mesh=mesh,
        scratch_types=[
            pltpu.VMEM((num_idx,), jnp.int32),
            pltpu.VMEM((slab,), jnp.float32),
        ],
        compiler_params=pltpu.CompilerParams(needs_layout_passes=False),
    )
    def mask_kernel(idx_hbm, mask_hbm, idx_v, slab_v):
        wid = lax.axis_index("s") * _NC + lax.axis_index("c")
        lo = wid * slab
        pltpu.sync_copy(idx_hbm, idx_v)

        zeros = jnp.zeros((_L,), jnp.float32)

        def zero_body(i, carry):
            slab_v[pl.ds(i * _L, _L)] = zeros
            return carry

        lax.fori_loop(0, slab // _L, zero_body, 0)

        ones = jnp.ones((_L,), jnp.float32)

        def scan_body(i, carry):
            v = idx_v[pl.ds(i * _L, _L)]
            m = (v >= lo) & (v < lo + slab)
            rel = jnp.where(m, v - lo, 0)
            plsc.store_scatter(slab_v, (rel,), ones, mask=m)
            return carry

        lax.fori_loop(0, num_idx // _L, scan_body, 0)

        pltpu.sync_copy(slab_v, mask_hbm.at[pl.ds(lo, slab)])

    return mask_kernel, m_pad


def _select_body(x_ref, m_ref, v_ref, o_ref):
    o_ref[...] = jnp.where(m_ref[...] != 0.0, v_ref[0, 0], x_ref[...])


def _select_tc(x, mask2d, value_f32, blk_rows: int):
    num_rows, d = x.shape
    grid = (pl.cdiv(num_rows, blk_rows),)
    return pl.pallas_call(
        _select_body,
        grid=grid,
        in_specs=[
            pl.BlockSpec((blk_rows, d), lambda i: (i, 0)),
            pl.BlockSpec((blk_rows, 1), lambda i: (i, 0)),
            pl.BlockSpec(memory_space=pltpu.SMEM),
        ],
        out_specs=pl.BlockSpec((blk_rows, d), lambda i: (i, 0)),
        out_shape=jax.ShapeDtypeStruct((num_rows, d), x.dtype),
    )(x, mask2d, value_f32)


def kernel(x, dim, index, value):
    num_rows, _ = x.shape
    num_idx = index.shape[0]
    idx32 = index.astype(jnp.int32)

    mask_fn, m_pad = _build_mask_sc(num_rows, num_idx)
    mask = mask_fn(idx32)
    mask2d = mask.reshape(m_pad, 1)

    value_f32 = jnp.full((1, 1), value, dtype=jnp.float32)
    return _select_tc(x, mask2d, value_f32, blk_rows=4000)


# speedup vs baseline: 1.4054x; 1.0282x over previous
import functools

import jax
import jax.numpy as jnp
from jax import lax
from jax.experimental import pallas as pl
from jax.experimental.pallas import tpu as pltpu
from jax.experimental.pallas import tpu_sc as plsc

_NC = 2
_NS = 16
_NW = _NC * _NS
_L = 16


def _build_mask_sc(num_rows: int, num_idx: int):
    slab = ((num_rows + _NW - 1) // _NW + _L - 1) // _L * _L
    m_pad = slab * _NW
    mesh = plsc.VectorSubcoreMesh(core_axis_name="c", subcore_axis_name="s")

    @functools.partial(
        pl.kernel,
        out_type=jax.ShapeDtypeStruct((m_pad,), jnp.float32),
        mesh=mesh,
        scratch_types=[
            pltpu.VMEM((num_idx,), jnp.int32),
            pltpu.VMEM((slab,), jnp.float32),
        ],
        compiler_params=pltpu.CompilerParams(needs_layout_passes=False),
    )
    def mask_kernel(idx_hbm, mask_hbm, idx_v, slab_v):
        wid = lax.axis_index("s") * _NC + lax.axis_index("c")
        lo = wid * slab
        pltpu.sync_copy(idx_hbm, idx_v)

        zeros = jnp.zeros((_L,), jnp.float32)

        def zero_body(i, carry):
            slab_v[pl.ds(i * _L, _L)] = zeros
            return carry

        lax.fori_loop(0, slab // _L, zero_body, 0)

        ones = jnp.ones((_L,), jnp.float32)

        def scan_body(i, carry):
            v = idx_v[pl.ds(i * _L, _L)]
            m = (v >= lo) & (v < lo + slab)
            rel = jnp.where(m, v - lo, 0)
            plsc.store_scatter(slab_v, (rel,), ones, mask=m)
            return carry

        lax.fori_loop(0, num_idx // _L, scan_body, 0)

        pltpu.sync_copy(slab_v, mask_hbm.at[pl.ds(lo, slab)])

    return mask_kernel, m_pad


def _select_body(x_ref, m_ref, v_ref, o_ref):
    o_ref[...] = jnp.where(m_ref[...] != 0.0, v_ref[0, 0], x_ref[...])


def _select_tc(x, mask2d, value_f32, blk_rows: int):
    num_rows, d = x.shape
    grid = (pl.cdiv(num_rows, blk_rows),)
    return pl.pallas_call(
        _select_body,
        grid=grid,
        in_specs=[
            pl.BlockSpec((blk_rows, d), lambda i: (i, 0)),
            pl.BlockSpec((blk_rows, 1), lambda i: (i, 0)),
            pl.BlockSpec(memory_space=pltpu.SMEM),
        ],
        out_specs=pl.BlockSpec((blk_rows, d), lambda i: (i, 0)),
        out_shape=jax.ShapeDtypeStruct((num_rows, d), x.dtype),
    )(x, mask2d, value_f32)


def kernel(x, dim, index, value):
    num_rows, _ = x.shape
    num_idx = index.shape[0]
    idx32 = index.astype(jnp.int32)

    mask_fn, m_pad = _build_mask_sc(num_rows, num_idx)
    mask = mask_fn(idx32)
    mask2d = mask.reshape(m_pad, 1)

    value_f32 = jnp.full((1, 1), value, dtype=jnp.float32)
    return _select_tc(x, mask2d, value_f32, blk_rows=10000)
